# baseline (speedup 1.0000x reference)
import functools

import jax
import jax.numpy as jnp
from jax import lax
from jax.experimental import pallas as pl
from jax.experimental.pallas import tpu as pltpu
from jax.experimental.pallas import tpu_sc as plsc


_NC = 2
_NS = 16
_LANES = 16


def _gelu(t):
    return 0.5 * t * (1.0 + lax.erf(t * 0.7071067811865476))


def _unperm_matrix(d):
    import numpy as np
    P = np.zeros((d, d), np.float32)
    for g in range(d // 32):
        for k in range(16):
            P[32 * g + k, 32 * g + 2 * k] = 1.0
            P[32 * g + 16 + k, 32 * g + 2 * k + 1] = 1.0
    return P


def _layer_norm(h, g, b):
    mu = jnp.mean(h, axis=-1, keepdims=True)
    var = jnp.mean((h - mu) ** 2, axis=-1, keepdims=True)
    return (h - mu) * lax.rsqrt(var + 1e-5) * g + b


def _encode_body(ntab, vocab, idx_ref, tab_ref, lng_ref, lnb_ref,
                 w1_ref, b1_ref, w2_ref, b2_ref, *out_refs):
    be = idx_ref.shape[2]
    d = tab_ref.shape[2]
    idx = idx_ref[0]
    acc = jnp.zeros((be, d), jnp.float32)
    for i in range(ntab):
        io = lax.broadcasted_iota(jnp.int32, (vocab, be), 0)
        oht = (idx[i:i + 1, :] == io).astype(jnp.bfloat16)
        acc = acc + lax.dot_general(
            oht, tab_ref[i].astype(jnp.bfloat16), (((0,), (0,)), ((), ())),
            preferred_element_type=jnp.float32)
    hn = _layer_norm(acc, lng_ref[0], lnb_ref[0])
    t = jnp.dot(hn.astype(jnp.bfloat16), w1_ref[...].astype(jnp.bfloat16),
                preferred_element_type=jnp.float32)
    t = _gelu(t + b1_ref[0])
    out = jnp.dot(t.astype(jnp.bfloat16), w2_ref[...].astype(jnp.bfloat16),
                  preferred_element_type=jnp.float32) + b2_ref[0]
    out_refs[0][...] = out


def _encode(idx, emb, lng, lnb, w1, b1, w2, b2, block, emit=('f32',)):
    del emit
    n, ntab = idx.shape
    _, vocab, d = emb.shape
    nblk = n // block
    idx_t = jnp.zeros((8, n), jnp.int32).at[:ntab].set(idx.T.astype(jnp.int32))
    idx_t = idx_t.reshape(8, nblk, block).transpose(1, 0, 2)
    out_shapes = [jax.ShapeDtypeStruct((n, d), jnp.float32)]
    out_specs = [pl.BlockSpec((block, d), lambda i: (i, 0))]
    res = pl.pallas_call(
        functools.partial(_encode_body, ntab, vocab),
        grid=(nblk,),
        in_specs=[
            pl.BlockSpec((1, 8, block), lambda i: (i, 0, 0)),
            pl.BlockSpec((ntab, vocab, d), lambda i: (0, 0, 0)),
            pl.BlockSpec((1, d), lambda i: (0, 0)),
            pl.BlockSpec((1, d), lambda i: (0, 0)),
            pl.BlockSpec((d, d), lambda i: (0, 0)),
            pl.BlockSpec((1, d), lambda i: (0, 0)),
            pl.BlockSpec((d, d), lambda i: (0, 0)),
            pl.BlockSpec((1, d), lambda i: (0, 0)),
        ],
        out_specs=out_specs,
        out_shape=out_shapes,
    )(idx_t, emb, lng.reshape(1, d), lnb.reshape(1, d), w1,
      b1.reshape(1, d), w2, b2.reshape(1, d))
    return res[0]


def _node_body(h_ref, p_ref, w1_ref, b1_ref, w2_ref, b2_ref, g_ref, b_ref,
               out_ref):
    h = h_ref[...]
    z = h + p_ref[0] + p_ref[1]
    t = jnp.dot(z, w1_ref[...], preferred_element_type=jnp.float32)
    t = jnp.maximum(t + b1_ref[0], 0.0)
    t = jnp.dot(t, w2_ref[...], preferred_element_type=jnp.float32)
    t = _gelu(t + b2_ref[0])
    out_ref[...] = _layer_norm(t + h, g_ref[0], b_ref[0])


def _node_update(h, parts, w1, b1, w2, b2, g, b, block):
    n, d = h.shape
    nblk = n // block
    return pl.pallas_call(
        _node_body,
        grid=(nblk,),
        in_specs=[
            pl.BlockSpec((block, d), lambda i: (i, 0)),
            pl.BlockSpec((2, block, d), lambda i: (0, i, 0)),
            pl.BlockSpec((d, d), lambda i: (0, 0)),
            pl.BlockSpec((1, d), lambda i: (0, 0)),
            pl.BlockSpec((d, d), lambda i: (0, 0)),
            pl.BlockSpec((1, d), lambda i: (0, 0)),
            pl.BlockSpec((1, d), lambda i: (0, 0)),
            pl.BlockSpec((1, d), lambda i: (0, 0)),
        ],
        out_specs=pl.BlockSpec((block, d), lambda i: (i, 0)),
        out_shape=jax.ShapeDtypeStruct((n, d), jnp.float32),
    )(h, parts, w1, b1.reshape(1, d), w2, b2.reshape(1, d),
      g.reshape(1, d), b.reshape(1, d))


def _make_msg_kernel(n, e_total, d, chunk, nch):
    nw = _NC * _NS
    epw = e_total // nw
    rpt = (n // _NS) // 8 * 8
    tail = n - rpt * _NS

    mesh = plsc.VectorSubcoreMesh(
        core_axis_name="c", subcore_axis_name="s",
        num_cores=_NC, num_subcores=_NS)

    @functools.partial(
        pl.kernel,
        out_type=jax.ShapeDtypeStruct((_NC, n, d), jnp.float32),
        mesh=mesh,
        scratch_types=[
            pltpu.VMEM((3, 2, chunk), jnp.int32),
            pltpu.VMEM((2, chunk, d), jnp.float32),
            pltpu.VMEM((2, chunk, d), jnp.float32),
            pltpu.VMEM_SHARED((n + 16, d), jnp.float32),
            pltpu.SemaphoreType.DMA,
            pltpu.SemaphoreType.DMA,
            pltpu.SemaphoreType.DMA,
        ],
    )
    def msg_kernel(h_hbm, e_hbm, idx_hbm, zero_hbm, out_hbm,
                   idx_v, hbuf, ebuf, acc_sh, sem_i, sem_g, sem_s):
        c = lax.axis_index("c")
        s = lax.axis_index("s")
        wid = s * _NC + c
        base = wid * epw

        pltpu.sync_copy(zero_hbm.at[pl.ds(s * rpt, rpt)],
                        acc_sh.at[pl.ds(s * rpt, rpt)])
        if tail:
            @pl.when(s == _NS - 1)
            def _():
                pltpu.sync_copy(zero_hbm.at[pl.ds(rpt * _NS, tail)],
                                acc_sh.at[pl.ds(rpt * _NS, tail)])

        def start_idx(j, q):
            pltpu.async_copy(idx_hbm.at[wid, j], idx_v.at[q], sem_i)

        def wait_idx(j, q):
            pltpu.make_async_copy(idx_hbm.at[wid, j], idx_v.at[q],
                                  sem_i).wait()

        def start_loads(j, q, p):
            pltpu.async_copy(h_hbm.at[idx_v.at[q, 0]], hbuf.at[p], sem_g)
            pltpu.async_copy(e_hbm.at[pl.ds(base + j * chunk, chunk)],
                             ebuf.at[p], sem_g)

        def wait_loads(j, q, p):
            pltpu.make_async_copy(h_hbm.at[idx_v.at[q, 0]], hbuf.at[p],
                                  sem_g).wait()
            pltpu.make_async_copy(e_hbm.at[pl.ds(base + j * chunk, chunk)],
                                  ebuf.at[p], sem_g).wait()

        def start_scatter(q, p):
            pltpu.async_copy(hbuf.at[p], acc_sh.at[idx_v.at[q, 1]], sem_s,
                             add=True)

        def wait_scatter(q, p):
            pltpu.make_async_copy(hbuf.at[p], acc_sh.at[idx_v.at[q, 1]],
                                  sem_s).wait()

        plsc.subcore_barrier()
        pltpu.sync_copy(idx_hbm.at[wid, 0], idx_v.at[0])
        start_idx(1, 1)
        start_loads(0, 0, 0)

        assert nch % 2 == 0

        def chunk_step(j, p):
            q = lax.rem(j, 3)
            wait_loads(j, q, p)

            @pl.when(j >= 1)
            def _():
                wait_scatter(lax.rem(j - 1, 3), 1 - p)

            @pl.when(j + 2 < nch)
            def _():
                start_idx(j + 2, lax.rem(j + 2, 3))

            @pl.when(j + 1 < nch)
            def _():
                q1 = lax.rem(j + 1, 3)
                wait_idx(j + 1, q1)
                start_loads(j + 1, q1, 1 - p)

            for r in range(chunk):
                for g in range(d // _LANES):
                    sl = pl.ds(g * _LANES, _LANES)
                    v = hbuf[p, r, sl] + ebuf[p, r, sl]
                    hbuf[p, r, sl] = jnp.maximum(v, 0.0)
            start_scatter(q, p)

        def pair_body(jj, carry):
            chunk_step(2 * jj, 0)
            chunk_step(2 * jj + 1, 1)
            return carry

        lax.fori_loop(0, nch // 2, pair_body, 0, unroll=False)
        wait_scatter(lax.rem(nch - 1, 3), 1)
        plsc.subcore_barrier()
        pltpu.sync_copy(acc_sh.at[pl.ds(s * rpt, rpt)],
                        out_hbm.at[c, pl.ds(s * rpt, rpt)])
        if tail:
            @pl.when(s == _NS - 1)
            def _():
                pltpu.sync_copy(acc_sh.at[pl.ds(rpt * _NS, tail)],
                                out_hbm.at[c, pl.ds(rpt * _NS, tail)])

    return msg_kernel


def kernel(x, edge_index, edge_attr, atom_emb, atom_ln_g, atom_ln_b, atom_W1,
           atom_b1, atom_W2, atom_b2, bond_emb, bond_ln_g, bond_ln_b, bond_W1,
           bond_b1, bond_W2, bond_b2, conv_W1, conv_b1, conv_W2, conv_b2,
           ln_g, ln_b):
    n = x.shape[0]
    e_total = edge_index.shape[1]
    d = atom_emb.shape[2]
    n_layers = conv_W1.shape[0]

    h = _encode(x, atom_emb, atom_ln_g, atom_ln_b, atom_W1, atom_b1,
                atom_W2, atom_b2, block=1000)
    block_e = 1000
    e_rows = -(-(e_total + 64) // block_e) * block_e
    ea_pad = jnp.pad(edge_attr.astype(jnp.int32),
                     ((0, e_rows - e_total), (0, 0)))
    e = _encode(ea_pad, bond_emb, bond_ln_g, bond_ln_b, bond_W1,
                bond_b1, bond_W2, bond_b2, block=block_e)

    chunk = 40
    nw = _NC * _NS
    epw = e_total // nw
    nch = -(-epw // chunk)
    pad = nch * chunk - epw
    src = edge_index[0].astype(jnp.int32).reshape(nw, epw)
    dst = edge_index[1].astype(jnp.int32).reshape(nw, epw)
    src = jnp.pad(src, ((0, 0), (0, pad)))
    dst = jnp.pad(dst, ((0, 0), (0, pad)), constant_values=n)
    idx = jnp.stack([src.reshape(nw, nch, chunk),
                     dst.reshape(nw, nch, chunk)], axis=2)
    zeros = jnp.zeros((n, d), jnp.float32)

    msg = _make_msg_kernel(n, e_total, d, chunk=chunk, nch=nch)

    for l in range(n_layers):
        parts = msg(h, e, idx, zeros)
        h = _node_update(h, parts, conv_W1[l], conv_b1[l], conv_W2[l],
                         conv_b2[l], ln_g[l], ln_b[l], block=1000)
    return h


# speedup vs baseline: 1.2760x; 1.2760x over previous
import functools

import jax
import jax.numpy as jnp
from jax import lax
from jax.experimental import pallas as pl
from jax.experimental.pallas import tpu as pltpu
from jax.experimental.pallas import tpu_sc as plsc


_NC = 2
_NS = 16
_LANES = 16


def _gelu(t):
    return 0.5 * t * (1.0 + lax.erf(t * 0.7071067811865476))


def _unperm_matrix(d):
    import numpy as np
    P = np.zeros((d, d), np.float32)
    for g in range(d // 32):
        for k in range(16):
            P[32 * g + k, 32 * g + 2 * k] = 1.0
            P[32 * g + 16 + k, 32 * g + 2 * k + 1] = 1.0
    return P


def _layer_norm(h, g, b):
    mu = jnp.mean(h, axis=-1, keepdims=True)
    var = jnp.mean((h - mu) ** 2, axis=-1, keepdims=True)
    return (h - mu) * lax.rsqrt(var + 1e-5) * g + b


def _encode_body(ntab, vocab, idx_ref, tab_ref, lng_ref, lnb_ref,
                 w1_ref, b1_ref, w2_ref, b2_ref, *out_refs):
    be = idx_ref.shape[2]
    d = tab_ref.shape[2]
    idx = idx_ref[0]
    acc = jnp.zeros((be, d), jnp.float32)
    for i in range(ntab):
        io = lax.broadcasted_iota(jnp.int32, (vocab, be), 0)
        oht = (idx[i:i + 1, :] == io).astype(jnp.float32)
        acc = acc + lax.dot_general(
            oht, tab_ref[i], (((0,), (0,)), ((), ())),
            preferred_element_type=jnp.float32)
    hn = _layer_norm(acc, lng_ref[0], lnb_ref[0])
    t = jnp.dot(hn, w1_ref[...], preferred_element_type=jnp.float32)
    t = _gelu(t + b1_ref[0])
    out = jnp.dot(t, w2_ref[...],
                  preferred_element_type=jnp.float32) + b2_ref[0]
    out_refs[0][...] = out


def _encode(idx, emb, lng, lnb, w1, b1, w2, b2, block, emit=('f32',)):
    del emit
    n, ntab = idx.shape
    _, vocab, d = emb.shape
    nblk = n // block
    idx_t = jnp.zeros((8, n), jnp.int32).at[:ntab].set(idx.T.astype(jnp.int32))
    idx_t = idx_t.reshape(8, nblk, block).transpose(1, 0, 2)
    out_shapes = [jax.ShapeDtypeStruct((n, d), jnp.float32)]
    out_specs = [pl.BlockSpec((block, d), lambda i: (i, 0))]
    res = pl.pallas_call(
        functools.partial(_encode_body, ntab, vocab),
        grid=(nblk,),
        in_specs=[
            pl.BlockSpec((1, 8, block), lambda i: (i, 0, 0)),
            pl.BlockSpec((ntab, vocab, d), lambda i: (0, 0, 0)),
            pl.BlockSpec((1, d), lambda i: (0, 0)),
            pl.BlockSpec((1, d), lambda i: (0, 0)),
            pl.BlockSpec((d, d), lambda i: (0, 0)),
            pl.BlockSpec((1, d), lambda i: (0, 0)),
            pl.BlockSpec((d, d), lambda i: (0, 0)),
            pl.BlockSpec((1, d), lambda i: (0, 0)),
        ],
        out_specs=out_specs,
        out_shape=out_shapes,
    )(idx_t, emb, lng.reshape(1, d), lnb.reshape(1, d), w1,
      b1.reshape(1, d), w2, b2.reshape(1, d))
    return res[0]


def _node_body(h_ref, p_ref, w1_ref, b1_ref, w2_ref, b2_ref, g_ref, b_ref,
               out_ref):
    h = h_ref[...]
    z = h + p_ref[0] + p_ref[1]
    t = jnp.dot(z, w1_ref[...], preferred_element_type=jnp.float32)
    t = jnp.maximum(t + b1_ref[0], 0.0)
    t = jnp.dot(t, w2_ref[...], preferred_element_type=jnp.float32)
    t = _gelu(t + b2_ref[0])
    out_ref[...] = _layer_norm(t + h, g_ref[0], b_ref[0])


def _node_update(h, parts, w1, b1, w2, b2, g, b, block):
    n, d = h.shape
    nblk = n // block
    return pl.pallas_call(
        _node_body,
        grid=(nblk,),
        in_specs=[
            pl.BlockSpec((block, d), lambda i: (i, 0)),
            pl.BlockSpec((2, block, d), lambda i: (0, i, 0)),
            pl.BlockSpec((d, d), lambda i: (0, 0)),
            pl.BlockSpec((1, d), lambda i: (0, 0)),
            pl.BlockSpec((d, d), lambda i: (0, 0)),
            pl.BlockSpec((1, d), lambda i: (0, 0)),
            pl.BlockSpec((1, d), lambda i: (0, 0)),
            pl.BlockSpec((1, d), lambda i: (0, 0)),
        ],
        out_specs=pl.BlockSpec((block, d), lambda i: (i, 0)),
        out_shape=jax.ShapeDtypeStruct((n, d), jnp.float32),
    )(h, parts, w1, b1.reshape(1, d), w2, b2.reshape(1, d),
      g.reshape(1, d), b.reshape(1, d))


def _make_msg_kernel(n, e_total, d, chunk, nch):
    nw = _NC * _NS
    epw = e_total // nw
    rpt = (n // _NS) // 8 * 8
    tail = n - rpt * _NS

    mesh = plsc.VectorSubcoreMesh(
        core_axis_name="c", subcore_axis_name="s",
        num_cores=_NC, num_subcores=_NS)

    @functools.partial(
        pl.kernel,
        out_type=jax.ShapeDtypeStruct((_NC, n, d), jnp.float32),
        mesh=mesh,
        scratch_types=[
            pltpu.VMEM((3, 3, chunk), jnp.int32),
            pltpu.VMEM((2, chunk, d), jnp.float32),
            pltpu.VMEM((2, chunk, d), jnp.float32),
            pltpu.VMEM_SHARED((n + 16, d), jnp.float32),
            pltpu.SemaphoreType.DMA,
            pltpu.SemaphoreType.DMA,
            pltpu.SemaphoreType.DMA,
        ],
    )
    def msg_kernel(h_hbm, e_hbm, idx_hbm, zero_hbm, out_hbm,
                   idx_v, hbuf, ebuf, acc_sh, sem_i, sem_g, sem_s):
        c = lax.axis_index("c")
        s = lax.axis_index("s")
        wid = s * _NC + c
        base = wid * epw

        pltpu.sync_copy(zero_hbm.at[pl.ds(s * rpt, rpt)],
                        acc_sh.at[pl.ds(s * rpt, rpt)])
        if tail:
            @pl.when(s == _NS - 1)
            def _():
                pltpu.sync_copy(zero_hbm.at[pl.ds(rpt * _NS, tail)],
                                acc_sh.at[pl.ds(rpt * _NS, tail)])

        def start_idx(j, q):
            pltpu.async_copy(idx_hbm.at[wid, j], idx_v.at[q], sem_i)

        def wait_idx(j, q):
            pltpu.make_async_copy(idx_hbm.at[wid, j], idx_v.at[q],
                                  sem_i).wait()

        def start_loads(j, q, p):
            pltpu.async_copy(h_hbm.at[idx_v.at[q, 0]], hbuf.at[p], sem_g)
            pltpu.async_copy(e_hbm.at[idx_v.at[q, 2]], ebuf.at[p], sem_g)

        def wait_loads(j, q, p):
            pltpu.make_async_copy(h_hbm.at[idx_v.at[q, 0]], hbuf.at[p],
                                  sem_g).wait()
            pltpu.make_async_copy(e_hbm.at[idx_v.at[q, 2]], ebuf.at[p],
                                  sem_g).wait()

        def start_scatter(q, p):
            pltpu.async_copy(hbuf.at[p], acc_sh.at[idx_v.at[q, 1]], sem_s,
                             add=True)

        def wait_scatter(q, p):
            pltpu.make_async_copy(hbuf.at[p], acc_sh.at[idx_v.at[q, 1]],
                                  sem_s).wait()

        plsc.subcore_barrier()
        pltpu.sync_copy(idx_hbm.at[wid, 0], idx_v.at[0])
        start_idx(1, 1)
        start_loads(0, 0, 0)

        assert nch % 2 == 0

        def chunk_step(j, p):
            q = lax.rem(j, 3)
            wait_loads(j, q, p)

            @pl.when(j >= 1)
            def _():
                wait_scatter(lax.rem(j - 1, 3), 1 - p)

            @pl.when(j + 2 < nch)
            def _():
                start_idx(j + 2, lax.rem(j + 2, 3))

            @pl.when(j + 1 < nch)
            def _():
                q1 = lax.rem(j + 1, 3)
                wait_idx(j + 1, q1)
                start_loads(j + 1, q1, 1 - p)

            for r in range(chunk):
                for g in range(d // _LANES):
                    sl = pl.ds(g * _LANES, _LANES)
                    v = hbuf[p, r, sl] + ebuf[p, r, sl]
                    hbuf[p, r, sl] = jnp.maximum(v, 0.0)
            start_scatter(q, p)

        def pair_body(jj, carry):
            chunk_step(2 * jj, 0)
            chunk_step(2 * jj + 1, 1)
            return carry

        lax.fori_loop(0, nch // 2, pair_body, 0, unroll=False)
        wait_scatter(lax.rem(nch - 1, 3), 1)
        plsc.subcore_barrier()
        pltpu.sync_copy(acc_sh.at[pl.ds(s * rpt, rpt)],
                        out_hbm.at[c, pl.ds(s * rpt, rpt)])
        if tail:
            @pl.when(s == _NS - 1)
            def _():
                pltpu.sync_copy(acc_sh.at[pl.ds(rpt * _NS, tail)],
                                out_hbm.at[c, pl.ds(rpt * _NS, tail)])

    return msg_kernel


def kernel(x, edge_index, edge_attr, atom_emb, atom_ln_g, atom_ln_b, atom_W1,
           atom_b1, atom_W2, atom_b2, bond_emb, bond_ln_g, bond_ln_b, bond_W1,
           bond_b1, bond_W2, bond_b2, conv_W1, conv_b1, conv_W2, conv_b2,
           ln_g, ln_b):
    n = x.shape[0]
    e_total = edge_index.shape[1]
    d = atom_emb.shape[2]
    n_layers = conv_W1.shape[0]

    h = _encode(x, atom_emb, atom_ln_g, atom_ln_b, atom_W1, atom_b1,
                atom_W2, atom_b2, block=1000)
    nb = edge_attr.shape[1]
    bv = bond_emb.shape[1]
    ncombo = bv ** nb
    ar = jnp.arange(ncombo, dtype=jnp.int32)
    combo = jnp.stack([(ar // (bv ** i)) % bv for i in range(nb)], axis=1)
    e = _encode(combo, bond_emb, bond_ln_g, bond_ln_b, bond_W1,
                bond_b1, bond_W2, bond_b2, block=1024)

    chunk = 40
    nw = _NC * _NS
    epw = e_total // nw
    nch = -(-epw // chunk)
    pad = nch * chunk - epw
    ea = edge_attr.astype(jnp.int32)
    eid = ea[:, 0] + bv * ea[:, 1] + bv * bv * ea[:, 2]
    src = edge_index[0].astype(jnp.int32).reshape(nw, epw)
    dst = edge_index[1].astype(jnp.int32).reshape(nw, epw)
    eid = eid.reshape(nw, epw)
    src = jnp.pad(src, ((0, 0), (0, pad)))
    dst = jnp.pad(dst, ((0, 0), (0, pad)), constant_values=n)
    eid = jnp.pad(eid, ((0, 0), (0, pad)))
    idx = jnp.stack([src.reshape(nw, nch, chunk),
                     dst.reshape(nw, nch, chunk),
                     eid.reshape(nw, nch, chunk)], axis=2)
    zeros = jnp.zeros((n, d), jnp.float32)

    msg = _make_msg_kernel(n, e_total, d, chunk=chunk, nch=nch)

    for l in range(n_layers):
        parts = msg(h, e, idx, zeros)
        h = _node_update(h, parts, conv_W1[l], conv_b1[l], conv_W2[l],
                         conv_b2[l], ln_g[l], ln_b[l], block=1000)
    return h
